# baseline (speedup 1.0000x reference)
import functools

import jax
import jax.numpy as jnp
from jax import lax
from jax.experimental import pallas as pl
from jax.experimental.pallas import tpu as pltpu
from jax.experimental.pallas import tpu_sc as plsc

NC = 2
NS = 16
NW = NC * NS
L = 16
CB = 4096
UNR = 8
FPW = 2


def _build(B, W, Fm1, V1):
  NCH = B // CB
  assert W == FPW * NW and B % CB == 0 and CB % (L * UNR) == 0

  mesh = plsc.VectorSubcoreMesh(core_axis_name="c", subcore_axis_name="s")

  @functools.partial(
      pl.kernel,
      out_type=jax.ShapeDtypeStruct((W, B), jnp.float32),
      mesh=mesh,
      scratch_types=[
          pltpu.VMEM((1, 1, V1), jnp.float32),
          pltpu.VMEM((1, B), jnp.float32),
          pltpu.VMEM((1, CB), jnp.int32),
          pltpu.VMEM((1, CB), jnp.int32),
          pltpu.VMEM((Fm1, L), jnp.float32),
          pltpu.SemaphoreType.DMA,
          pltpu.SemaphoreType.DMA,
          pltpu.SemaphoreType.DMA,
      ],
      compiler_params=pltpu.CompilerParams(
          use_tc_tiling_on_sc=True, needs_layout_passes=False),
  )
  def kern(e0r, tabs, xT, zb, out,
           col_v, out_v, idx0, idx1, zb_v, csem, isem0, isem1):
    wid = lax.axis_index("s") * NC + lax.axis_index("c")
    zz = jnp.zeros((L,), jnp.int32)
    ibufs = (idx0, idx1)
    isems = (isem0, isem1)

    pltpu.sync_copy(zb, zb_v)

    def column_pass(col_src, ridx, w):
      ch = pltpu.async_copy(col_src, col_v, csem)
      handles = {0: pltpu.async_copy(
          xT.at[pl.ds(ridx, 1), pl.ds(0, CB)], ibufs[0], isems[0])}
      ch.wait()
      for c in range(NCH):
        if c + 1 < NCH:
          nxt = (c + 1) % 2
          handles[c + 1] = pltpu.async_copy(
              xT.at[pl.ds(ridx, 1), pl.ds((c + 1) * CB, CB)],
              ibufs[nxt], isems[nxt])
        handles[c].wait()
        buf = ibufs[c % 2]

        def body(g, carry, c=c, buf=buf):
          for u in range(UNR):
            k = g * (L * UNR) + u * L
            v = buf[0, pl.ds(k, L)]
            val = plsc.load_gather(col_v, [zz, zz, v])
            boff = c * CB + k
            if w is None:
              out_v[0, pl.ds(boff, L)] = val
            else:
              out_v[0, pl.ds(boff, L)] = out_v[0, pl.ds(boff, L)] + w * val
          return carry

        lax.fori_loop(0, CB // (L * UNR), body, 0)

    for f_sel in range(FPW):
      f = wid * FPW + f_sel
      column_pass(e0r.at[pl.ds(0, 1), pl.ds(f, 1), pl.ds(0, V1)], 0, None)

      def tbody(i, carry, f=f):
        wrow = plsc.load_gather(
            zb_v, [jnp.full((L,), i, jnp.int32), lax.iota(jnp.int32, L)])
        w = 0.5 * jnp.exp(wrow)
        column_pass(
            tabs.at[pl.ds(i, 1), pl.ds(f, 1), pl.ds(0, V1)], i + 1, w)
        return carry

      lax.fori_loop(0, Fm1, tbody, 0)
      pltpu.sync_copy(out_v, out.at[pl.ds(f, 1), pl.ds(0, B)])

  return kern


@jax.jit
def kernel(x, embed0, tables, zero):
  B, F = x.shape
  V1, W = embed0.shape
  Fm1 = F - 1

  tabs = tables.transpose(0, 2, 1)
  e0r = embed0.T.reshape(1, W, V1)
  xT = x.T
  zb = jnp.broadcast_to(zero[:, None], (Fm1, L))

  kern = _build(B, W, Fm1, V1)
  outT = kern(e0r, tabs, xT, zb)
  return outT.T
